# Initial kernel scaffold; baseline (speedup 1.0000x reference)
import jax
import jax.numpy as jnp
from jax.experimental import pallas as pl


def kernel(memory, spatial_shapes, level_start_index, Wc, bc, W1, b1, W2, b2):
    raise NotImplementedError("write your pallas kernel here")


# speedup vs baseline: 1.0705x; 1.0705x over previous
import functools

import jax
import jax.numpy as jnp
from jax import lax
from jax.experimental import pallas as pl
from jax.experimental.pallas import tpu as pltpu
from jax.experimental.pallas import tpu_sc as plsc

NUM_QUERIES = 900
D_MODEL = 1024
NUM_CLASSES = 91
BS = 4
SEQ = 8192
N_ROWS = BS * SEQ
SLOTS = 1024
BLK_M = 512


def _score_body(x_ref, wc_ref, w1_ref, w2_ref, maxc_ref, t2_ref, t3_ref):
    x = x_ref[...]
    logits = jnp.dot(x, wc_ref[...], preferred_element_type=jnp.float32)
    mask = jax.lax.broadcasted_iota(jnp.int32, logits.shape, 1) < NUM_CLASSES
    logits = jnp.where(mask, logits, -jnp.inf)
    maxc_ref[...] = jnp.max(logits, axis=-1)
    h = jax.nn.relu(jnp.dot(x, w1_ref[...], preferred_element_type=jnp.float32))
    t = jnp.dot(h, w2_ref[...], preferred_element_type=jnp.float32)
    t2_ref[...] = t[:, 2]
    t3_ref[...] = t[:, 3]


def _scores_pallas(memory, Wc, W1, W2):
    x = memory.reshape(N_ROWS, D_MODEL)
    wc_pad = jnp.zeros((D_MODEL, 128), jnp.float32).at[:, :NUM_CLASSES].set(Wc)
    w2_pad = jnp.zeros((D_MODEL, 128), jnp.float32).at[:, :4].set(W2)
    maxc, t2, t3 = pl.pallas_call(
        _score_body,
        grid=(N_ROWS // BLK_M,),
        in_specs=[
            pl.BlockSpec((BLK_M, D_MODEL), lambda i: (i, 0)),
            pl.BlockSpec((D_MODEL, 128), lambda i: (0, 0)),
            pl.BlockSpec((D_MODEL, D_MODEL), lambda i: (0, 0)),
            pl.BlockSpec((D_MODEL, 128), lambda i: (0, 0)),
        ],
        out_specs=[
            pl.BlockSpec((BLK_M,), lambda i: (i,)),
            pl.BlockSpec((BLK_M,), lambda i: (i,)),
            pl.BlockSpec((BLK_M,), lambda i: (i,)),
        ],
        out_shape=[
            jax.ShapeDtypeStruct((N_ROWS,), jnp.float32),
            jax.ShapeDtypeStruct((N_ROWS,), jnp.float32),
            jax.ShapeDtypeStruct((N_ROWS,), jnp.float32),
        ],
    )(x, wc_pad, W1, w2_pad)
    return (maxc.reshape(BS, SEQ), t2.reshape(BS, SEQ), t3.reshape(BS, SEQ))


def _topk_body(s_ref, out_ref):
    bits = jax.lax.bitcast_convert_type(s_ref[0], jnp.int32)

    def count_ge(t):
        return jnp.sum((bits >= t).astype(jnp.int32), axis=1, keepdims=True)

    def step(_, lohi):
        lo, hi = lohi
        mid = lo + ((hi - lo) >> 1)
        ge = count_ge(mid) >= NUM_QUERIES
        return (jnp.where(ge, mid, lo), jnp.where(ge, hi, mid))

    lo = jnp.zeros((1, 1), jnp.int32)
    hi = jnp.full((1, 1), 1 << 30, jnp.int32)
    lo, hi = jax.lax.fori_loop(0, 31, step, (lo, hi))
    thr = lo

    cand = bits >= thr
    s = cand.astype(jnp.int32)
    z = jnp.zeros_like(s)
    k = 1
    while k < SEQ:
        s = s + jnp.concatenate([z[:, :k], s[:, :-k]], axis=1)
        k *= 2
    pos = s - 1

    slot_iota = jax.lax.broadcasted_iota(jnp.int32, (SLOTS, SLOTS), 0)
    col_iota = jax.lax.broadcasted_iota(jnp.int32, (1, SLOTS), 1)

    comp = jnp.zeros((SLOTS, 6), jnp.float32)
    for c in range(SEQ // SLOTS):
        sl = slice(c * SLOTS, (c + 1) * SLOTS)
        p = pos[:, sl]
        m = cand[:, sl]
        bt = bits[:, sl]
        col = col_iota + c * SLOTS
        oh = jnp.where((slot_iota == p) & m, 1.0, 0.0)
        vals = jnp.concatenate(
            [((col >> 8) & 0xFF).astype(jnp.float32),
             (col & 0xFF).astype(jnp.float32),
             ((bt >> 24) & 0xFF).astype(jnp.float32),
             ((bt >> 16) & 0xFF).astype(jnp.float32),
             ((bt >> 8) & 0xFF).astype(jnp.float32),
             (bt & 0xFF).astype(jnp.float32)], axis=0)
        comp = comp + jax.lax.dot_general(
            oh, vals, (((1,), (1,)), ((), ())),
            preferred_element_type=jnp.float32)

    idx = (comp[:, 0].astype(jnp.int32) << 8) | comp[:, 1].astype(jnp.int32)
    cbits = ((comp[:, 2].astype(jnp.int32) << 24)
             | (comp[:, 3].astype(jnp.int32) << 16)
             | (comp[:, 4].astype(jnp.int32) << 8)
             | comp[:, 5].astype(jnp.int32))
    gt = (cbits[None, :] > cbits[:, None])
    tie = (cbits[None, :] == cbits[:, None]) & (idx[None, :] < idx[:, None])
    rank = jnp.sum((gt | tie).astype(jnp.int32), axis=1)
    ohr = jnp.where(slot_iota == rank[None, :], 1.0, 0.0)
    idx2 = jnp.concatenate(
        [((idx >> 8) & 0xFF)[None, :].astype(jnp.float32),
         (idx & 0xFF)[None, :].astype(jnp.float32)], axis=0)
    srt = jax.lax.dot_general(
        ohr, idx2, (((1,), (1,)), ((), ())),
        preferred_element_type=jnp.float32)
    out_ref[0, 0, :] = (srt[:, 0].astype(jnp.int32) << 8) | srt[:, 1].astype(jnp.int32)


def _topk_pallas(scores):
    out = pl.pallas_call(
        _topk_body,
        grid=(BS,),
        in_specs=[pl.BlockSpec((1, 1, SEQ), lambda b: (b, 0, 0))],
        out_specs=pl.BlockSpec((1, 1, SLOTS), lambda b: (b, 0, 0)),
        out_shape=jax.ShapeDtypeStruct((BS, 1, SLOTS), jnp.int32),
    )(scores.reshape(BS, 1, SEQ))
    return out.reshape(BS, SLOTS)


def _gather_sc(table, flat_idx):
    info = plsc.get_sparse_core_info()
    nw = info.num_cores * info.num_subcores
    n = flat_idx.shape[0]
    b_per_w = n // nw
    chunk = 64
    nchunk = b_per_w // chunk
    mesh = plsc.VectorSubcoreMesh(core_axis_name="c", subcore_axis_name="s")

    @functools.partial(
        pl.kernel, mesh=mesh,
        out_type=jax.ShapeDtypeStruct((n, D_MODEL), jnp.float32),
        scratch_types=[
            pltpu.VMEM((nchunk, chunk), jnp.int32),
            pltpu.VMEM((chunk, D_MODEL), jnp.float32),
            pltpu.SemaphoreType.DMA,
        ],
    )
    def gk(table_hbm, idx_hbm, out_hbm, idx_v, rows_v, sem):
        wid = lax.axis_index("s") * info.num_cores + lax.axis_index("c")
        pltpu.sync_copy(idx_hbm.at[wid], idx_v)
        for j in range(nchunk):
            pltpu.async_copy(table_hbm.at[idx_v.at[j]], rows_v, sem).wait()
            pltpu.sync_copy(
                rows_v, out_hbm.at[pl.ds(wid * b_per_w + j * chunk, chunk)])

    return gk(table, flat_idx.reshape(nw, nchunk, chunk))


def kernel(memory, spatial_shapes, level_start_index, Wc, bc, W1, b1, W2, b2):
    output_class = memory @ Wc + bc
    h = jax.nn.relu(memory @ W1 + b1)
    output_coord = jax.nn.sigmoid(h @ W2 + b2)
    cls_scores = jnp.max(jax.nn.sigmoid(output_class), axis=-1)
    iou_scores = output_coord[..., 2] * output_coord[..., 3]
    scores = 1.0 - jnp.abs(cls_scores - iou_scores)

    _, topk_indices = jax.lax.top_k(scores, NUM_QUERIES)
    pad = jnp.broadcast_to(
        jnp.arange(SLOTS - NUM_QUERIES, dtype=jnp.int32)[None, :],
        (BS, SLOTS - NUM_QUERIES))
    padded = jnp.concatenate([topk_indices, pad], axis=1)

    flat = (padded + (jnp.arange(BS, dtype=jnp.int32) * SEQ)[:, None]).reshape(-1)
    rows = _gather_sc(memory.reshape(N_ROWS, D_MODEL), flat)
    selected_memory = rows.reshape(BS, SLOTS, D_MODEL)[:, :NUM_QUERIES, :]
    return (selected_memory, topk_indices)
